# Initial kernel scaffold; baseline (speedup 1.0000x reference)
import jax
import jax.numpy as jnp
from jax.experimental import pallas as pl


def kernel(input):
    raise NotImplementedError("write your pallas kernel here")


# speedup vs baseline: 36.3481x; 36.3481x over previous
import numpy as np
import jax
import jax.numpy as jnp
from jax.experimental import pallas as pl

_GH, _GW, _NB, _NC = 52, 52, 5, 80
_N = _GH * _GW * _NB
_ROWS, _LANES = 106, 128
_NPAD = _ROWS * _LANES
_MAX_OUT = 100
_IOU_THR = 0.5
_CONF_THR = 0.01

_ANCHORS = np.array(
    [0.57273, 0.677385, 1.87446, 2.06253, 3.33843, 5.47434,
     7.88282, 3.52778, 9.77052, 9.16828], dtype=np.float32).reshape(_NB, 2)


def _build_consts():
    p = np.arange(_NPAD)
    gx = (p // _NB) % _GW
    gy = p // (_NB * _GW)
    a = p % _NB
    cellx = gx.astype(np.float32)
    celly = gy.astype(np.float32)
    aw = _ANCHORS[a, 0]
    ah = _ANCHORS[a, 1]
    validf = (p < _N).astype(np.float32)
    cst = np.stack([cellx, celly, aw, ah, validf], axis=0)
    return cst.reshape(5, _ROWS, _LANES)

_CONSTS = _build_consts()


def _yolo_nms_kernel(x5_ref, cls_ref, cst_ref, out_ref):
    f32 = jnp.float32
    gwf = f32(_GW)

    tx = x5_ref[0]
    ty = x5_ref[1]
    tw = x5_ref[2]
    th = x5_ref[3]
    tc = x5_ref[4]
    cellx = cst_ref[0]
    celly = cst_ref[1]
    aw = cst_ref[2]
    ah = cst_ref[3]
    vmask = cst_ref[4]

    cx = (jax.nn.sigmoid(tx) + cellx) / gwf
    cy = (jax.nn.sigmoid(ty) + celly) / gwf
    bw = (jnp.exp(tw) * aw) / gwf
    bh = (jnp.exp(th) * ah) / gwf
    x1 = cx - bw / 2.0
    y1 = cy - bh / 2.0
    x2 = cx + bw / 2.0
    y2 = cy + bh / 2.0
    areas = (x2 - x1) * (y2 - y1)

    cls = cls_ref[...]
    mlog = jnp.max(cls, axis=0)
    ssum = jnp.sum(jnp.exp(cls - mlog[None]), axis=0)
    iota80 = jax.lax.broadcasted_iota(
        jnp.int32, (_NC, _ROWS, _LANES), 0).astype(f32)
    clsid = jnp.min(jnp.where(cls == mlog[None], iota80, f32(1e9)), axis=0)
    conf_raw = jax.nn.sigmoid(tc) * (1.0 / ssum)
    conf = jnp.where(conf_raw > _CONF_THR, conf_raw, 0.0)
    s0 = jnp.where(vmask > 0, conf, -jnp.inf)

    iota_r = jax.lax.broadcasted_iota(jnp.int32, (_ROWS, _LANES), 0)
    iota_c = jax.lax.broadcasted_iota(jnp.int32, (_ROWS, _LANES), 1)
    iota_flat = (iota_r * _LANES + iota_c).astype(f32)
    lane_iota = jax.lax.broadcasted_iota(jnp.int32, (1, _LANES), 1)
    neg = f32(-jnp.inf)
    zrow = jnp.zeros((1, _LANES), f32)

    def body(i, s):
        m = jnp.max(s)
        idxf = jnp.min(jnp.where(s == m, iota_flat, f32(1e9)))
        valid = m > 0.0
        validf = jnp.where(valid, f32(1.0), f32(0.0))
        oh = iota_flat == idxf
        bx1 = jnp.sum(jnp.where(oh, x1, 0.0))
        by1 = jnp.sum(jnp.where(oh, y1, 0.0))
        bx2 = jnp.sum(jnp.where(oh, x2, 0.0))
        by2 = jnp.sum(jnp.where(oh, y2, 0.0))
        bcls = jnp.sum(jnp.where(oh, clsid, 0.0))
        ix1 = jnp.maximum(x1, bx1)
        iy1 = jnp.maximum(y1, by1)
        ix2 = jnp.minimum(x2, bx2)
        iy2 = jnp.minimum(y2, by2)
        inter = jnp.maximum(ix2 - ix1, 0.0) * jnp.maximum(iy2 - iy1, 0.0)
        barea = (bx2 - bx1) * (by2 - by1)
        iou = inter / (areas + barea - inter + 1e-9)
        supp = (iou > _IOU_THR) & valid
        s_new = jnp.where(supp | oh, neg, s)
        lane_oh = lane_iota == i
        upd = jnp.concatenate([
            jnp.where(lane_oh, m * validf, 0.0),
            jnp.where(lane_oh, bx1 * validf, 0.0),
            jnp.where(lane_oh, by1 * validf, 0.0),
            jnp.where(lane_oh, bx2 * validf, 0.0),
            jnp.where(lane_oh, by2 * validf, 0.0),
            jnp.where(lane_oh, bcls * validf, 0.0),
            zrow, zrow], axis=0)
        out_ref[...] += upd
        return s_new

    out_ref[...] = jnp.zeros((8, _LANES), f32)
    jax.lax.fori_loop(0, _MAX_OUT, body, s0)


def _run(x5, clst, interpret=False):
    return pl.pallas_call(
        _yolo_nms_kernel,
        out_shape=jax.ShapeDtypeStruct((8, _LANES), jnp.float32),
        interpret=interpret,
    )(x5, clst, jnp.asarray(_CONSTS))


def kernel(input):
    xt = input.reshape(_N, 85).T
    xt = jnp.pad(xt, ((0, 0), (0, _NPAD - _N)))
    x5 = xt[:5].reshape(5, _ROWS, _LANES)
    clst = xt[5:].reshape(_NC, _ROWS, _LANES)
    out = _run(x5, clst)
    conf = out[0, :_MAX_OUT]
    boxes = out[1:5, :_MAX_OUT].T
    classes = out[5, :_MAX_OUT].astype(jnp.int32)
    return conf, boxes, classes
